# baseline (speedup 1.0000x reference)
import functools

import jax
import jax.numpy as jnp
from jax import lax
from jax.experimental import pallas as pl
from jax.experimental.pallas import tpu as pltpu
from jax.experimental.pallas import tpu_sc as plsc

_EPS = 1e-5

_N = 10000
_E = 320000
_D = 128

_NC = 2
_NS = 16
_NW = _NC * _NS
_L = 16

_EW = _E // _NW
_C = 80
_NCHUNK = _EW // _C
_NP = 10240
_NSTRIPE = _NP // _NS
_ZR = 80


def _leaky(x):
    return jnp.maximum(x, 0.2 * x)


def _sc_gather_body(u_hbm, v_hbm, src_hbm, dst_hbm,
                    ud_hbm, vs_hbm, p_hbm, histd_hbm, hists_hbm,
                    dix, six, rows_u, rows_v, histd, hists,
                    p_sh, sem_u, sem_v):
    c = lax.axis_index("c")
    s = lax.axis_index("s")
    wid = s * _NC + c

    def fill_zrow(i, _):
        for j in range(_D // _L):
            rows_u[i, pl.ds(j * _L, _L)] = jnp.zeros((_L,), jnp.float32)
        return 0

    lax.fori_loop(0, _ZR, fill_zrow, 0)

    def fill_hist(i, _):
        histd[pl.ds(i * _L, _L)] = jnp.zeros((_L,), jnp.float32)
        hists[pl.ds(i * _L, _L)] = jnp.zeros((_L,), jnp.float32)
        return 0

    lax.fori_loop(0, _NP // _L, fill_hist, 0)

    row0 = s * _NSTRIPE
    for k in range(_NSTRIPE // _ZR):
        pltpu.sync_copy(rows_u, p_sh.at[pl.ds(row0 + k * _ZR, _ZR)])

    plsc.subcore_barrier()

    base0 = wid * _EW
    ones16 = jnp.ones((_L,), jnp.float32)

    def chunk(i, _):
        base = base0 + i * _C
        pltpu.sync_copy(dst_hbm.at[pl.ds(base, _C)], dix)
        pltpu.sync_copy(src_hbm.at[pl.ds(base, _C)], six)
        cp_u = pltpu.async_copy(u_hbm.at[dix], rows_u, sem_u)
        cp_v = pltpu.async_copy(v_hbm.at[six], rows_v, sem_v)
        for g in range(_C // _L):
            dv = dix[pl.ds(g * _L, _L)]
            sv = six[pl.ds(g * _L, _L)]
            plsc.addupdate_scatter(histd, [dv], ones16)
            plsc.addupdate_scatter(hists, [sv], ones16)
        cp_u.wait()
        cp_v.wait()
        pltpu.sync_copy(rows_u, ud_hbm.at[pl.ds(base, _C)])
        pltpu.sync_copy(rows_v, vs_hbm.at[pl.ds(base, _C)])
        pltpu.sync_copy(rows_v, p_sh.at[dix], add=True)
        return 0

    lax.fori_loop(0, _NCHUNK, chunk, 0)

    plsc.subcore_barrier()

    pltpu.sync_copy(p_sh.at[pl.ds(row0, _NSTRIPE)], p_hbm.at[c, pl.ds(row0, _NSTRIPE)])
    pltpu.sync_copy(histd, histd_hbm.at[wid])
    pltpu.sync_copy(hists, hists_hbm.at[wid])


def _sc_gather(u, v, src, dst):
    mesh = plsc.VectorSubcoreMesh(core_axis_name="c", subcore_axis_name="s")
    fn = functools.partial(
        pl.kernel,
        mesh=mesh,
        compiler_params=pltpu.CompilerParams(needs_layout_passes=False),
        out_type=[
            jax.ShapeDtypeStruct((_E, _D), jnp.float32),
            jax.ShapeDtypeStruct((_E, _D), jnp.float32),
            jax.ShapeDtypeStruct((_NC, _NP, _D), jnp.float32),
            jax.ShapeDtypeStruct((_NW, _NP), jnp.float32),
            jax.ShapeDtypeStruct((_NW, _NP), jnp.float32),
        ],
        scratch_types=[
            pltpu.VMEM((_C,), jnp.int32),
            pltpu.VMEM((_C,), jnp.int32),
            pltpu.VMEM((_C, _D), jnp.float32),
            pltpu.VMEM((_C, _D), jnp.float32),
            pltpu.VMEM((_NP,), jnp.float32),
            pltpu.VMEM((_NP,), jnp.float32),
            pltpu.VMEM_SHARED((_NP, _D), jnp.float32),
            pltpu.SemaphoreType.DMA,
            pltpu.SemaphoreType.DMA,
        ],
    )(_sc_gather_body)
    return fn(u, v, src, dst)


_FW = _D // _NW
_CH = 640
_NCHR = _E // _CH
_GK = 4
_EB4 = _CH * _GK


def _sc_scatter_max_body(y2tr_hbm, dst_hbm, srep_hbm, trep_hbm,
                         agg_hbm,
                         dstb, idxb, vals, st_v, acc, sem_g):
    c = lax.axis_index("c")
    s = lax.axis_index("s")
    wid = s * _NC + c
    f0 = wid * _FW

    neg_inf = jnp.full((_L,), -jnp.inf, jnp.float32)

    def fill_acc(i, _):
        for j in range(_FW):
            acc[j, pl.ds(i * _L, _L)] = neg_inf
        return 0

    lax.fori_loop(0, _NP // _L, fill_acc, 0)

    pltpu.sync_copy(srep_hbm.at[wid], st_v.at[0])
    pltpu.sync_copy(trep_hbm.at[wid], st_v.at[1])

    lanes = lax.iota(jnp.int32, _L)
    jlane = lax.shift_right_logical(lanes, 2)
    ilane = lanes & 3

    def chunk(kk, _):
        c0 = kk * _GK
        e0 = c0 * _CH
        pltpu.sync_copy(dst_hbm.at[pl.ds(e0, _EB4)], dstb)
        idxb[...] = (f0 + jlane) * _NCHR + c0 + ilane
        pltpu.async_copy(y2tr_hbm.at[idxb], vals, sem_g).wait()

        for j in range(_FW):
            sj = st_v[0, j, pl.ds(0, _L)]
            tj = st_v[1, j, pl.ds(0, _L)]
            jv = jnp.full((_L,), j, jnp.int32)

            for i in range(_GK):
                def group(g, _):
                    dstv = dstb[pl.ds(i * _CH + g * _L, _L)]
                    val = vals[j * _GK + i, pl.ds(g * _L, _L)]
                    val = val * sj + tj
                    val = jnp.maximum(val, 0.2 * val)
                    cur = plsc.load_gather(acc, [jv, dstv])
                    m = val > cur

                    def cond(mm):
                        return jnp.any(mm)

                    def body(mm):
                        plsc.store_scatter(acc, [jv, dstv], val, mask=mm)
                        cur2 = plsc.load_gather(acc, [jv, dstv])
                        return val > cur2

                    lax.while_loop(cond, body, m)
                    return 0

                lax.fori_loop(0, _CH // _L, group, 0)
        return 0

    lax.fori_loop(0, _NCHR // _GK, chunk, 0)

    pltpu.sync_copy(acc, agg_hbm.at[wid])


def _sc_scatter_max(y2tr, dst, srep, trep):
    mesh = plsc.VectorSubcoreMesh(core_axis_name="c", subcore_axis_name="s")
    fn = functools.partial(
        pl.kernel,
        mesh=mesh,
        compiler_params=pltpu.CompilerParams(needs_layout_passes=False),
        out_type=[
            jax.ShapeDtypeStruct((_NW, _FW, _NP), jnp.float32),
        ],
        scratch_types=[
            pltpu.VMEM((_EB4,), jnp.int32),
            pltpu.VMEM((_L,), jnp.int32),
            pltpu.VMEM((_L, _CH), jnp.float32),
            pltpu.VMEM((2, _FW, _D), jnp.float32),
            pltpu.VMEM((_FW, _NP), jnp.float32),
            pltpu.SemaphoreType.DMA,
        ],
    )(_sc_scatter_max_body)
    return fn(y2tr, dst, srep, trep)[0]


_BN0 = 2000
_BE = 1280


def _k0_body(x_ref, wa_ref, wb_ref, u_ref, v_ref):
    x = x_ref[...]
    u_ref[...] = jnp.dot(x, wa_ref[...], preferred_element_type=jnp.float32)
    v_ref[...] = jnp.dot(x, wb_ref[...], preferred_element_type=jnp.float32)


def _k0(x, wa, wb):
    return pl.pallas_call(
        _k0_body,
        grid=(_N // _BN0,),
        in_specs=[
            pl.BlockSpec((_BN0, _D), lambda i: (i, 0)),
            pl.BlockSpec((_D, _D), lambda i: (0, 0)),
            pl.BlockSpec((_D, _D), lambda i: (0, 0)),
        ],
        out_specs=[
            pl.BlockSpec((_BN0, _D), lambda i: (i, 0)),
            pl.BlockSpec((_BN0, _D), lambda i: (i, 0)),
        ],
        out_shape=[
            jax.ShapeDtypeStruct((_N, _D), jnp.float32),
            jax.ShapeDtypeStruct((_N, _D), jnp.float32),
        ],
    )(x, wa, wb)


def _k3_body(ud_ref, vs_ref, s1_ref, t1_ref, w2_ref, y2t_ref, ss_ref, sq_ref):
    y1 = ud_ref[...] + vs_ref[...]
    h2 = y1 * s1_ref[...] + t1_ref[...]
    h2 = jnp.maximum(h2, 0.2 * h2)
    y2t = jax.lax.dot_general(w2_ref[...], h2, (((1,), (1,)), ((), ())),
                              preferred_element_type=jnp.float32)
    y2t_ref[...] = y2t

    @pl.when(pl.program_id(0) == 0)
    def _():
        ss_ref[...] = jnp.zeros_like(ss_ref)
        sq_ref[...] = jnp.zeros_like(sq_ref)

    ss_ref[...] += jnp.sum(y2t, axis=1)[None, :]
    sq_ref[...] += jnp.sum(y2t * y2t, axis=1)[None, :]


def _k3(ud, vs, s1, t1, w2):
    return pl.pallas_call(
        _k3_body,
        grid=(_E // _BE,),
        in_specs=[
            pl.BlockSpec((_BE, _D), lambda i: (i, 0)),
            pl.BlockSpec((_BE, _D), lambda i: (i, 0)),
            pl.BlockSpec((1, _D), lambda i: (0, 0)),
            pl.BlockSpec((1, _D), lambda i: (0, 0)),
            pl.BlockSpec((_D, _D), lambda i: (0, 0)),
        ],
        out_specs=[
            pl.BlockSpec((_D, _BE), lambda i: (0, i)),
            pl.BlockSpec((1, _D), lambda i: (0, 0)),
            pl.BlockSpec((1, _D), lambda i: (0, 0)),
        ],
        out_shape=[
            jax.ShapeDtypeStruct((_D, _E), jnp.float32),
            jax.ShapeDtypeStruct((1, _D), jnp.float32),
            jax.ShapeDtypeStruct((1, _D), jnp.float32),
        ],
    )(ud, vs, s1, t1, w2)


def _k5_body(agg_ref, x_ref, o_ref):
    a = agg_ref[...].T
    a = jnp.where(jnp.isfinite(a), a, 0.0)
    o = a + x_ref[...]
    o_ref[...] = jnp.maximum(o, 0.2 * o)


_BN5 = 2048


def _k5(agg, xp):
    return pl.pallas_call(
        _k5_body,
        grid=(_NP // _BN5,),
        in_specs=[
            pl.BlockSpec((_D, _BN5), lambda i: (0, i)),
            pl.BlockSpec((_BN5, _D), lambda i: (i, 0)),
        ],
        out_specs=pl.BlockSpec((_BN5, _D), lambda i: (i, 0)),
        out_shape=jax.ShapeDtypeStruct((_NP, _D), jnp.float32),
    )(agg, xp)


def kernel(x, edge_index, W1, g1, b1, W2, g2, b2):
    N, D = x.shape
    E = edge_index.shape[1]
    src = edge_index[0]
    dst = edge_index[1]

    wa = (W1[:, :D] - W1[:, D:]).T
    wb = W1[:, D:].T
    u, v = _k0(x, wa, wb)

    ud, vs, p_parts, histd, hists = _sc_gather(u, v, src, dst)
    P = p_parts[0, :N] + p_parts[1, :N]
    cnt_dst = jnp.sum(histd, axis=0)[:N]
    cnt_src = jnp.sum(hists, axis=0)[:N]

    sum1 = cnt_dst @ u + cnt_src @ v
    sq1 = cnt_dst @ (u * u) + 2.0 * jnp.sum(u * P, axis=0) + cnt_src @ (v * v)
    mean1 = sum1 / E
    var1 = sq1 / E - mean1 * mean1
    s1 = g1 / jnp.sqrt(var1 + _EPS)
    t1 = b1 - mean1 * s1

    y2t, ssum, ssq = _k3(ud, vs, s1[None, :], t1[None, :], W2)
    mean2 = ssum[0] / E
    var2 = ssq[0] / E - mean2 * mean2
    s2 = g2 / jnp.sqrt(var2 + _EPS)
    t2 = b2 - mean2 * s2

    srep = jnp.broadcast_to(s2.reshape(_NW, _FW, 1), (_NW, _FW, _D))
    trep = jnp.broadcast_to(t2.reshape(_NW, _FW, 1), (_NW, _FW, _D))
    y2tr = y2t.reshape(_D * _NCHR, _CH)
    agg = _sc_scatter_max(y2tr, dst, srep, trep).reshape(_D, _NP)

    xp = jnp.pad(x, ((0, _NP - N), (0, 0)))
    return _k5(agg, xp)[:N]


# speedup vs baseline: 1.7122x; 1.7122x over previous
import functools

import jax
import jax.numpy as jnp
from jax import lax
from jax.experimental import pallas as pl
from jax.experimental.pallas import tpu as pltpu
from jax.experimental.pallas import tpu_sc as plsc

_EPS = 1e-5

_N = 10000
_E = 320000
_D = 128

_NC = 2
_NS = 16
_NW = _NC * _NS
_L = 16

_EW = _E // _NW
_C = 80
_NCHUNK = _EW // _C
_NP = 10240
_NSTRIPE = _NP // _NS
_ZR = 80


def _leaky(x):
    return jnp.maximum(x, 0.2 * x)


def _sc_gather_body(u_hbm, v_hbm, src_hbm, dst_hbm,
                    ud_hbm, vs_hbm, p_hbm, histd_hbm, hists_hbm,
                    dix, six, rows_u, rows_v, histd, hists,
                    p_sh, sem_u, sem_v):
    c = lax.axis_index("c")
    s = lax.axis_index("s")
    wid = s * _NC + c

    def fill_zrow(i, _):
        for j in range(_D // _L):
            rows_u[i, pl.ds(j * _L, _L)] = jnp.zeros((_L,), jnp.float32)
        return 0

    lax.fori_loop(0, _ZR, fill_zrow, 0)

    def fill_hist(i, _):
        histd[pl.ds(i * _L, _L)] = jnp.zeros((_L,), jnp.float32)
        hists[pl.ds(i * _L, _L)] = jnp.zeros((_L,), jnp.float32)
        return 0

    lax.fori_loop(0, _NP // _L, fill_hist, 0)

    row0 = s * _NSTRIPE
    for k in range(_NSTRIPE // _ZR):
        pltpu.sync_copy(rows_u, p_sh.at[pl.ds(row0 + k * _ZR, _ZR)])

    plsc.subcore_barrier()

    base0 = wid * _EW
    ones16 = jnp.ones((_L,), jnp.float32)

    def chunk(i, _):
        base = base0 + i * _C
        pltpu.sync_copy(dst_hbm.at[pl.ds(base, _C)], dix)
        pltpu.sync_copy(src_hbm.at[pl.ds(base, _C)], six)
        cp_u = pltpu.async_copy(u_hbm.at[dix], rows_u, sem_u)
        cp_v = pltpu.async_copy(v_hbm.at[six], rows_v, sem_v)
        for g in range(_C // _L):
            dv = dix[pl.ds(g * _L, _L)]
            sv = six[pl.ds(g * _L, _L)]
            plsc.addupdate_scatter(histd, [dv], ones16)
            plsc.addupdate_scatter(hists, [sv], ones16)
        cp_u.wait()
        cp_v.wait()
        pltpu.sync_copy(rows_u, ud_hbm.at[pl.ds(base, _C)])
        pltpu.sync_copy(rows_v, vs_hbm.at[pl.ds(base, _C)])
        pltpu.sync_copy(rows_v, p_sh.at[dix], add=True)
        return 0

    lax.fori_loop(0, _NCHUNK, chunk, 0)

    plsc.subcore_barrier()

    pltpu.sync_copy(p_sh.at[pl.ds(row0, _NSTRIPE)], p_hbm.at[c, pl.ds(row0, _NSTRIPE)])
    pltpu.sync_copy(histd, histd_hbm.at[wid])
    pltpu.sync_copy(hists, hists_hbm.at[wid])


def _sc_gather(u, v, src, dst):
    mesh = plsc.VectorSubcoreMesh(core_axis_name="c", subcore_axis_name="s")
    fn = functools.partial(
        pl.kernel,
        mesh=mesh,
        compiler_params=pltpu.CompilerParams(needs_layout_passes=False),
        out_type=[
            jax.ShapeDtypeStruct((_E, _D), jnp.float32),
            jax.ShapeDtypeStruct((_E, _D), jnp.float32),
            jax.ShapeDtypeStruct((_NC, _NP, _D), jnp.float32),
            jax.ShapeDtypeStruct((_NW, _NP), jnp.float32),
            jax.ShapeDtypeStruct((_NW, _NP), jnp.float32),
        ],
        scratch_types=[
            pltpu.VMEM((_C,), jnp.int32),
            pltpu.VMEM((_C,), jnp.int32),
            pltpu.VMEM((_C, _D), jnp.float32),
            pltpu.VMEM((_C, _D), jnp.float32),
            pltpu.VMEM((_NP,), jnp.float32),
            pltpu.VMEM((_NP,), jnp.float32),
            pltpu.VMEM_SHARED((_NP, _D), jnp.float32),
            pltpu.SemaphoreType.DMA,
            pltpu.SemaphoreType.DMA,
        ],
    )(_sc_gather_body)
    return fn(u, v, src, dst)


_FW = _D // _NW
_CH = 640
_NCHR = _E // _CH
_GK = 4
_EB4 = _CH * _GK


def _group_pass(acc, dstb, vals, sj, tj, jv, row, off, col):
    dstv = dstb[pl.ds(off, _L)]
    val = vals[row, pl.ds(col, _L)]
    val = val * sj + tj
    val = jnp.maximum(val, 0.2 * val)
    cur = plsc.load_gather(acc, [jv, dstv])
    plsc.store_scatter(acc, [jv, dstv], val, mask=val > cur)
    cur2 = plsc.load_gather(acc, [jv, dstv])
    return val > cur2


def _group_fix(acc, dstb, vals, sj, tj, jv, row, off, col):
    dstv = dstb[pl.ds(off, _L)]
    val = vals[row, pl.ds(col, _L)]
    val = val * sj + tj
    val = jnp.maximum(val, 0.2 * val)
    cur = plsc.load_gather(acc, [jv, dstv])

    def cond(mm):
        return jnp.any(mm)

    def body(mm):
        plsc.store_scatter(acc, [jv, dstv], val, mask=mm)
        cur2 = plsc.load_gather(acc, [jv, dstv])
        return val > cur2

    lax.while_loop(cond, body, val > cur)


_UNROLL = 4


def _sc_scatter_max_body(y2tr_hbm, dst_hbm, srep_hbm, trep_hbm,
                         agg_hbm,
                         dstb, idxb, vals, st_v, acc, sem_g):
    c = lax.axis_index("c")
    s = lax.axis_index("s")
    wid = s * _NC + c
    f0 = wid * _FW

    neg_inf = jnp.full((_L,), -jnp.inf, jnp.float32)

    def fill_acc(i, _):
        for j in range(_FW):
            acc[j, pl.ds(i * _L, _L)] = neg_inf
        return 0

    lax.fori_loop(0, _NP // _L, fill_acc, 0)

    pltpu.sync_copy(srep_hbm.at[wid], st_v.at[0])
    pltpu.sync_copy(trep_hbm.at[wid], st_v.at[1])

    lanes = lax.iota(jnp.int32, _L)
    jlane = lax.shift_right_logical(lanes, 2)
    ilane = lanes & 3

    def chunk(kk, _):
        c0 = kk * _GK
        e0 = c0 * _CH
        pltpu.sync_copy(dst_hbm.at[pl.ds(e0, _EB4)], dstb)
        idxb[...] = (f0 + jlane) * _NCHR + c0 + ilane
        pltpu.async_copy(y2tr_hbm.at[idxb], vals, sem_g).wait()

        for j in range(_FW):
            sj = st_v[0, j, pl.ds(0, _L)]
            tj = st_v[1, j, pl.ds(0, _L)]
            jv = jnp.full((_L,), j, jnp.int32)
            row = j * _GK

            for i in range(_GK):
                def step(t, _):
                    lost = None
                    offs = []
                    for q in range(_UNROLL):
                        off = i * _CH + t * (_UNROLL * _L) + q * _L
                        col = t * (_UNROLL * _L) + q * _L
                        offs.append((off, col))
                        lq = _group_pass(acc, dstb, vals, sj, tj, jv,
                                         row + i, off, col)
                        lost = lq if lost is None else jnp.logical_or(lost, lq)

                    @pl.when(jnp.any(lost))
                    def _():
                        for off, col in offs:
                            _group_fix(acc, dstb, vals, sj, tj, jv,
                                       row + i, off, col)
                    return 0

                lax.fori_loop(0, _CH // (_UNROLL * _L), step, 0)
        return 0

    lax.fori_loop(0, _NCHR // _GK, chunk, 0)

    pltpu.sync_copy(acc, agg_hbm.at[wid])


def _sc_scatter_max(y2tr, dst, srep, trep):
    mesh = plsc.VectorSubcoreMesh(core_axis_name="c", subcore_axis_name="s")
    fn = functools.partial(
        pl.kernel,
        mesh=mesh,
        compiler_params=pltpu.CompilerParams(needs_layout_passes=False),
        out_type=[
            jax.ShapeDtypeStruct((_NW, _FW, _NP), jnp.float32),
        ],
        scratch_types=[
            pltpu.VMEM((_EB4,), jnp.int32),
            pltpu.VMEM((_L,), jnp.int32),
            pltpu.VMEM((_L, _CH), jnp.float32),
            pltpu.VMEM((2, _FW, _D), jnp.float32),
            pltpu.VMEM((_FW, _NP), jnp.float32),
            pltpu.SemaphoreType.DMA,
        ],
    )(_sc_scatter_max_body)
    return fn(y2tr, dst, srep, trep)[0]


_BN0 = 2000
_BE = 1280


def _k0_body(x_ref, wa_ref, wb_ref, u_ref, v_ref):
    x = x_ref[...]
    u_ref[...] = jnp.dot(x, wa_ref[...], preferred_element_type=jnp.float32)
    v_ref[...] = jnp.dot(x, wb_ref[...], preferred_element_type=jnp.float32)


def _k0(x, wa, wb):
    return pl.pallas_call(
        _k0_body,
        grid=(_N // _BN0,),
        in_specs=[
            pl.BlockSpec((_BN0, _D), lambda i: (i, 0)),
            pl.BlockSpec((_D, _D), lambda i: (0, 0)),
            pl.BlockSpec((_D, _D), lambda i: (0, 0)),
        ],
        out_specs=[
            pl.BlockSpec((_BN0, _D), lambda i: (i, 0)),
            pl.BlockSpec((_BN0, _D), lambda i: (i, 0)),
        ],
        out_shape=[
            jax.ShapeDtypeStruct((_N, _D), jnp.float32),
            jax.ShapeDtypeStruct((_N, _D), jnp.float32),
        ],
    )(x, wa, wb)


def _k3_body(ud_ref, vs_ref, s1_ref, t1_ref, w2_ref, y2t_ref, ss_ref, sq_ref):
    y1 = ud_ref[...] + vs_ref[...]
    h2 = y1 * s1_ref[...] + t1_ref[...]
    h2 = jnp.maximum(h2, 0.2 * h2)
    y2t = jax.lax.dot_general(w2_ref[...], h2, (((1,), (1,)), ((), ())),
                              preferred_element_type=jnp.float32)
    y2t_ref[...] = y2t

    @pl.when(pl.program_id(0) == 0)
    def _():
        ss_ref[...] = jnp.zeros_like(ss_ref)
        sq_ref[...] = jnp.zeros_like(sq_ref)

    ss_ref[...] += jnp.sum(y2t, axis=1)[None, :]
    sq_ref[...] += jnp.sum(y2t * y2t, axis=1)[None, :]


def _k3(ud, vs, s1, t1, w2):
    return pl.pallas_call(
        _k3_body,
        grid=(_E // _BE,),
        in_specs=[
            pl.BlockSpec((_BE, _D), lambda i: (i, 0)),
            pl.BlockSpec((_BE, _D), lambda i: (i, 0)),
            pl.BlockSpec((1, _D), lambda i: (0, 0)),
            pl.BlockSpec((1, _D), lambda i: (0, 0)),
            pl.BlockSpec((_D, _D), lambda i: (0, 0)),
        ],
        out_specs=[
            pl.BlockSpec((_D, _BE), lambda i: (0, i)),
            pl.BlockSpec((1, _D), lambda i: (0, 0)),
            pl.BlockSpec((1, _D), lambda i: (0, 0)),
        ],
        out_shape=[
            jax.ShapeDtypeStruct((_D, _E), jnp.float32),
            jax.ShapeDtypeStruct((1, _D), jnp.float32),
            jax.ShapeDtypeStruct((1, _D), jnp.float32),
        ],
    )(ud, vs, s1, t1, w2)


def _k5_body(agg_ref, x_ref, o_ref):
    a = agg_ref[...].T
    a = jnp.where(jnp.isfinite(a), a, 0.0)
    o = a + x_ref[...]
    o_ref[...] = jnp.maximum(o, 0.2 * o)


_BN5 = 2048


def _k5(agg, xp):
    return pl.pallas_call(
        _k5_body,
        grid=(_NP // _BN5,),
        in_specs=[
            pl.BlockSpec((_D, _BN5), lambda i: (0, i)),
            pl.BlockSpec((_BN5, _D), lambda i: (i, 0)),
        ],
        out_specs=pl.BlockSpec((_BN5, _D), lambda i: (i, 0)),
        out_shape=jax.ShapeDtypeStruct((_NP, _D), jnp.float32),
    )(agg, xp)


def kernel(x, edge_index, W1, g1, b1, W2, g2, b2):
    N, D = x.shape
    E = edge_index.shape[1]
    src = edge_index[0]
    dst = edge_index[1]

    wa = (W1[:, :D] - W1[:, D:]).T
    wb = W1[:, D:].T
    u, v = _k0(x, wa, wb)

    ud, vs, p_parts, histd, hists = _sc_gather(u, v, src, dst)
    P = p_parts[0, :N] + p_parts[1, :N]
    cnt_dst = jnp.sum(histd, axis=0)[:N]
    cnt_src = jnp.sum(hists, axis=0)[:N]

    sum1 = cnt_dst @ u + cnt_src @ v
    sq1 = cnt_dst @ (u * u) + 2.0 * jnp.sum(u * P, axis=0) + cnt_src @ (v * v)
    mean1 = sum1 / E
    var1 = sq1 / E - mean1 * mean1
    s1 = g1 / jnp.sqrt(var1 + _EPS)
    t1 = b1 - mean1 * s1

    y2t, ssum, ssq = _k3(ud, vs, s1[None, :], t1[None, :], W2)
    mean2 = ssum[0] / E
    var2 = ssq[0] / E - mean2 * mean2
    s2 = g2 / jnp.sqrt(var2 + _EPS)
    t2 = b2 - mean2 * s2

    srep = jnp.broadcast_to(s2.reshape(_NW, _FW, 1), (_NW, _FW, _D))
    trep = jnp.broadcast_to(t2.reshape(_NW, _FW, 1), (_NW, _FW, _D))
    y2tr = y2t.reshape(_D * _NCHR, _CH)
    agg = _sc_scatter_max(y2tr, dst, srep, trep).reshape(_D, _NP)

    xp = jnp.pad(x, ((0, _NP - N), (0, 0)))
    return _k5(agg, xp)[:N]


# speedup vs baseline: 2.3075x; 1.3476x over previous
import functools

import jax
import jax.numpy as jnp
from jax import lax
from jax.experimental import pallas as pl
from jax.experimental.pallas import tpu as pltpu
from jax.experimental.pallas import tpu_sc as plsc

_EPS = 1e-5

_N = 10000
_E = 320000
_D = 128

_NC = 2
_NS = 16
_NW = _NC * _NS
_L = 16

_EW = _E // _NW
_C = 80
_NCHUNK = _EW // _C
_NP = 10240
_NSTRIPE = _NP // _NS
_ZR = 80


def _leaky(x):
    return jnp.maximum(x, 0.2 * x)


def _sc_gather_body(u_hbm, v_hbm, src_hbm, dst_hbm,
                    ud_hbm, vs_hbm, p_hbm, histd_hbm, hists_hbm,
                    dix, six, rows_u, rows_v, histd, hists,
                    p_sh, sem_u, sem_v):
    c = lax.axis_index("c")
    s = lax.axis_index("s")
    wid = s * _NC + c

    def fill_zrow(i, _):
        for j in range(_D // _L):
            rows_u[i, pl.ds(j * _L, _L)] = jnp.zeros((_L,), jnp.float32)
        return 0

    lax.fori_loop(0, _ZR, fill_zrow, 0)

    def fill_hist(i, _):
        histd[pl.ds(i * _L, _L)] = jnp.zeros((_L,), jnp.float32)
        hists[pl.ds(i * _L, _L)] = jnp.zeros((_L,), jnp.float32)
        return 0

    lax.fori_loop(0, _NP // _L, fill_hist, 0)

    row0 = s * _NSTRIPE
    for k in range(_NSTRIPE // _ZR):
        pltpu.sync_copy(rows_u, p_sh.at[pl.ds(row0 + k * _ZR, _ZR)])

    plsc.subcore_barrier()

    base0 = wid * _EW
    ones16 = jnp.ones((_L,), jnp.float32)

    def chunk(i, _):
        base = base0 + i * _C
        pltpu.sync_copy(dst_hbm.at[pl.ds(base, _C)], dix)
        pltpu.sync_copy(src_hbm.at[pl.ds(base, _C)], six)
        cp_u = pltpu.async_copy(u_hbm.at[dix], rows_u, sem_u)
        cp_v = pltpu.async_copy(v_hbm.at[six], rows_v, sem_v)
        for g in range(_C // _L):
            dv = dix[pl.ds(g * _L, _L)]
            sv = six[pl.ds(g * _L, _L)]
            plsc.addupdate_scatter(histd, [dv], ones16)
            plsc.addupdate_scatter(hists, [sv], ones16)
        cp_u.wait()
        cp_v.wait()
        pltpu.sync_copy(rows_u, ud_hbm.at[pl.ds(base, _C)])
        pltpu.sync_copy(rows_v, vs_hbm.at[pl.ds(base, _C)])
        pltpu.sync_copy(rows_v, p_sh.at[dix], add=True)
        return 0

    lax.fori_loop(0, _NCHUNK, chunk, 0)

    plsc.subcore_barrier()

    pltpu.sync_copy(p_sh.at[pl.ds(row0, _NSTRIPE)], p_hbm.at[c, pl.ds(row0, _NSTRIPE)])
    pltpu.sync_copy(histd, histd_hbm.at[wid])
    pltpu.sync_copy(hists, hists_hbm.at[wid])


def _sc_gather(u, v, src, dst):
    mesh = plsc.VectorSubcoreMesh(core_axis_name="c", subcore_axis_name="s")
    fn = functools.partial(
        pl.kernel,
        mesh=mesh,
        compiler_params=pltpu.CompilerParams(needs_layout_passes=False),
        out_type=[
            jax.ShapeDtypeStruct((_E, _D), jnp.float32),
            jax.ShapeDtypeStruct((_E, _D), jnp.float32),
            jax.ShapeDtypeStruct((_NC, _NP, _D), jnp.float32),
            jax.ShapeDtypeStruct((_NW, _NP), jnp.float32),
            jax.ShapeDtypeStruct((_NW, _NP), jnp.float32),
        ],
        scratch_types=[
            pltpu.VMEM((_C,), jnp.int32),
            pltpu.VMEM((_C,), jnp.int32),
            pltpu.VMEM((_C, _D), jnp.float32),
            pltpu.VMEM((_C, _D), jnp.float32),
            pltpu.VMEM((_NP,), jnp.float32),
            pltpu.VMEM((_NP,), jnp.float32),
            pltpu.VMEM_SHARED((_NP, _D), jnp.float32),
            pltpu.SemaphoreType.DMA,
            pltpu.SemaphoreType.DMA,
        ],
    )(_sc_gather_body)
    return fn(u, v, src, dst)


_FW = _D // _NW
_CH = 640
_NCHR = _E // _CH
_GK = 4
_EB4 = _CH * _GK


def _group_fix(acc, dstb, vals, sj, tj, jv, row, off, col):
    dstv = dstb[pl.ds(off, _L)]
    val = vals[row, pl.ds(col, _L)]
    val = val * sj + tj
    val = jnp.maximum(val, 0.2 * val)
    cur = plsc.load_gather(acc, [jv, dstv])

    def cond(mm):
        return jnp.any(mm)

    def body(mm):
        plsc.store_scatter(acc, [jv, dstv], val, mask=mm)
        cur2 = plsc.load_gather(acc, [jv, dstv])
        return val > cur2

    lax.while_loop(cond, body, val > cur)


_UNROLL = 4


def _sc_scatter_max_body(y2tr_hbm, dst_hbm, srep_hbm, trep_hbm,
                         agg_hbm,
                         dstb, idxb, vals, st_v, acc, sem_g):
    c = lax.axis_index("c")
    s = lax.axis_index("s")
    wid = s * _NC + c
    f0 = wid * _FW

    neg_inf = jnp.full((_L,), -jnp.inf, jnp.float32)

    def fill_acc(i, _):
        for j in range(_FW):
            acc[j, pl.ds(i * _L, _L)] = neg_inf
        return 0

    lax.fori_loop(0, _NP // _L, fill_acc, 0)

    pltpu.sync_copy(srep_hbm.at[wid], st_v.at[0])
    pltpu.sync_copy(trep_hbm.at[wid], st_v.at[1])

    lanes = lax.iota(jnp.int32, _L)
    jlane = lax.shift_right_logical(lanes, 2)
    ilane = lanes & 3

    def chunk(kk, _):
        c0 = kk * _GK
        e0 = c0 * _CH
        pltpu.sync_copy(dst_hbm.at[pl.ds(e0, _EB4)], dstb)
        idxb[...] = (f0 + jlane) * _NCHR + c0 + ilane
        pltpu.async_copy(y2tr_hbm.at[idxb], vals, sem_g).wait()

        for j in range(_FW):
            sj = st_v[0, j, pl.ds(0, _L)]
            tj = st_v[1, j, pl.ds(0, _L)]
            jv = jnp.full((_L,), j, jnp.int32)
            row = j * _GK

            for i in range(_GK):
                def step(t, _):
                    offs, dsts, vs = [], [], []
                    for q in range(_UNROLL):
                        off = i * _CH + t * (_UNROLL * _L) + q * _L
                        col = t * (_UNROLL * _L) + q * _L
                        offs.append((off, col))
                        dstv = dstb[pl.ds(off, _L)]
                        val = vals[row + i, pl.ds(col, _L)]
                        val = val * sj + tj
                        val = jnp.maximum(val, 0.2 * val)
                        dsts.append(dstv)
                        vs.append(val)
                    curs = [plsc.load_gather(acc, [jv, d]) for d in dsts]
                    for q in range(_UNROLL):
                        plsc.store_scatter(acc, [jv, dsts[q]], vs[q],
                                           mask=vs[q] > curs[q])
                    lost = None
                    for q in range(_UNROLL):
                        cur2 = plsc.load_gather(acc, [jv, dsts[q]])
                        lq = vs[q] > cur2
                        lost = lq if lost is None else jnp.logical_or(lost, lq)

                    @pl.when(jnp.any(lost))
                    def _():
                        for off, col in offs:
                            _group_fix(acc, dstb, vals, sj, tj, jv,
                                       row + i, off, col)
                    return 0

                lax.fori_loop(0, _CH // (_UNROLL * _L), step, 0)
        return 0

    lax.fori_loop(0, _NCHR // _GK, chunk, 0)

    pltpu.sync_copy(acc, agg_hbm.at[wid])


def _sc_scatter_max(y2tr, dst, srep, trep):
    mesh = plsc.VectorSubcoreMesh(core_axis_name="c", subcore_axis_name="s")
    fn = functools.partial(
        pl.kernel,
        mesh=mesh,
        compiler_params=pltpu.CompilerParams(needs_layout_passes=False),
        out_type=[
            jax.ShapeDtypeStruct((_NW, _FW, _NP), jnp.float32),
        ],
        scratch_types=[
            pltpu.VMEM((_EB4,), jnp.int32),
            pltpu.VMEM((_L,), jnp.int32),
            pltpu.VMEM((_L, _CH), jnp.float32),
            pltpu.VMEM((2, _FW, _D), jnp.float32),
            pltpu.VMEM((_FW, _NP), jnp.float32),
            pltpu.SemaphoreType.DMA,
        ],
    )(_sc_scatter_max_body)
    return fn(y2tr, dst, srep, trep)[0]


_BN0 = 2000
_BE = 1280


def _k0_body(x_ref, wa_ref, wb_ref, u_ref, v_ref):
    x = x_ref[...]
    u_ref[...] = jnp.dot(x, wa_ref[...], preferred_element_type=jnp.float32)
    v_ref[...] = jnp.dot(x, wb_ref[...], preferred_element_type=jnp.float32)


def _k0(x, wa, wb):
    return pl.pallas_call(
        _k0_body,
        grid=(_N // _BN0,),
        in_specs=[
            pl.BlockSpec((_BN0, _D), lambda i: (i, 0)),
            pl.BlockSpec((_D, _D), lambda i: (0, 0)),
            pl.BlockSpec((_D, _D), lambda i: (0, 0)),
        ],
        out_specs=[
            pl.BlockSpec((_BN0, _D), lambda i: (i, 0)),
            pl.BlockSpec((_BN0, _D), lambda i: (i, 0)),
        ],
        out_shape=[
            jax.ShapeDtypeStruct((_N, _D), jnp.float32),
            jax.ShapeDtypeStruct((_N, _D), jnp.float32),
        ],
    )(x, wa, wb)


def _k3_body(ud_ref, vs_ref, s1_ref, t1_ref, w2_ref, y2t_ref, ss_ref, sq_ref):
    y1 = ud_ref[...] + vs_ref[...]
    h2 = y1 * s1_ref[...] + t1_ref[...]
    h2 = jnp.maximum(h2, 0.2 * h2)
    y2t = jax.lax.dot_general(w2_ref[...], h2, (((1,), (1,)), ((), ())),
                              preferred_element_type=jnp.float32)
    y2t_ref[...] = y2t

    @pl.when(pl.program_id(0) == 0)
    def _():
        ss_ref[...] = jnp.zeros_like(ss_ref)
        sq_ref[...] = jnp.zeros_like(sq_ref)

    ss_ref[...] += jnp.sum(y2t, axis=1)[None, :]
    sq_ref[...] += jnp.sum(y2t * y2t, axis=1)[None, :]


def _k3(ud, vs, s1, t1, w2):
    return pl.pallas_call(
        _k3_body,
        grid=(_E // _BE,),
        in_specs=[
            pl.BlockSpec((_BE, _D), lambda i: (i, 0)),
            pl.BlockSpec((_BE, _D), lambda i: (i, 0)),
            pl.BlockSpec((1, _D), lambda i: (0, 0)),
            pl.BlockSpec((1, _D), lambda i: (0, 0)),
            pl.BlockSpec((_D, _D), lambda i: (0, 0)),
        ],
        out_specs=[
            pl.BlockSpec((_D, _BE), lambda i: (0, i)),
            pl.BlockSpec((1, _D), lambda i: (0, 0)),
            pl.BlockSpec((1, _D), lambda i: (0, 0)),
        ],
        out_shape=[
            jax.ShapeDtypeStruct((_D, _E), jnp.float32),
            jax.ShapeDtypeStruct((1, _D), jnp.float32),
            jax.ShapeDtypeStruct((1, _D), jnp.float32),
        ],
    )(ud, vs, s1, t1, w2)


def _k5_body(agg_ref, x_ref, o_ref):
    a = agg_ref[...].T
    a = jnp.where(jnp.isfinite(a), a, 0.0)
    o = a + x_ref[...]
    o_ref[...] = jnp.maximum(o, 0.2 * o)


_BN5 = 2048


def _k5(agg, xp):
    return pl.pallas_call(
        _k5_body,
        grid=(_NP // _BN5,),
        in_specs=[
            pl.BlockSpec((_D, _BN5), lambda i: (0, i)),
            pl.BlockSpec((_BN5, _D), lambda i: (i, 0)),
        ],
        out_specs=pl.BlockSpec((_BN5, _D), lambda i: (i, 0)),
        out_shape=jax.ShapeDtypeStruct((_NP, _D), jnp.float32),
    )(agg, xp)


def kernel(x, edge_index, W1, g1, b1, W2, g2, b2):
    N, D = x.shape
    E = edge_index.shape[1]
    src = edge_index[0]
    dst = edge_index[1]

    wa = (W1[:, :D] - W1[:, D:]).T
    wb = W1[:, D:].T
    u, v = _k0(x, wa, wb)

    ud, vs, p_parts, histd, hists = _sc_gather(u, v, src, dst)
    P = p_parts[0, :N] + p_parts[1, :N]
    cnt_dst = jnp.sum(histd, axis=0)[:N]
    cnt_src = jnp.sum(hists, axis=0)[:N]

    sum1 = cnt_dst @ u + cnt_src @ v
    sq1 = cnt_dst @ (u * u) + 2.0 * jnp.sum(u * P, axis=0) + cnt_src @ (v * v)
    mean1 = sum1 / E
    var1 = sq1 / E - mean1 * mean1
    s1 = g1 / jnp.sqrt(var1 + _EPS)
    t1 = b1 - mean1 * s1

    y2t, ssum, ssq = _k3(ud, vs, s1[None, :], t1[None, :], W2)
    mean2 = ssum[0] / E
    var2 = ssq[0] / E - mean2 * mean2
    s2 = g2 / jnp.sqrt(var2 + _EPS)
    t2 = b2 - mean2 * s2

    srep = jnp.broadcast_to(s2.reshape(_NW, _FW, 1), (_NW, _FW, _D))
    trep = jnp.broadcast_to(t2.reshape(_NW, _FW, 1), (_NW, _FW, _D))
    y2tr = y2t.reshape(_D * _NCHR, _CH)
    agg = _sc_scatter_max(y2tr, dst, srep, trep).reshape(_D, _NP)

    xp = jnp.pad(x, ((0, _NP - N), (0, 0)))
    return _k5(agg, xp)[:N]


# speedup vs baseline: 2.4393x; 1.0572x over previous
import functools

import jax
import jax.numpy as jnp
from jax import lax
from jax.experimental import pallas as pl
from jax.experimental.pallas import tpu as pltpu
from jax.experimental.pallas import tpu_sc as plsc

_EPS = 1e-5

_N = 10000
_E = 320000
_D = 128

_NC = 2
_NS = 16
_NW = _NC * _NS
_L = 16

_EW = _E // _NW
_C = 80
_NCHUNK = _EW // _C
_NP = 10240
_NSTRIPE = _NP // _NS
_ZR = 80


def _leaky(x):
    return jnp.maximum(x, 0.2 * x)


def _sc_gather_body(u_hbm, v_hbm, src_hbm, dst_hbm,
                    ud_hbm, vs_hbm, p_hbm, histd_hbm, hists_hbm,
                    dix, six, rows_u, rows_v, histd, hists,
                    p_sh, sem_u, sem_v):
    c = lax.axis_index("c")
    s = lax.axis_index("s")
    wid = s * _NC + c

    def fill_zrow(i, _):
        for j in range(_D // _L):
            rows_u[i, pl.ds(j * _L, _L)] = jnp.zeros((_L,), jnp.float32)
        return 0

    lax.fori_loop(0, _ZR, fill_zrow, 0)

    def fill_hist(i, _):
        histd[pl.ds(i * _L, _L)] = jnp.zeros((_L,), jnp.float32)
        hists[pl.ds(i * _L, _L)] = jnp.zeros((_L,), jnp.float32)
        return 0

    lax.fori_loop(0, _NP // _L, fill_hist, 0)

    row0 = s * _NSTRIPE
    for k in range(_NSTRIPE // _ZR):
        pltpu.sync_copy(rows_u, p_sh.at[pl.ds(row0 + k * _ZR, _ZR)])

    plsc.subcore_barrier()

    base0 = wid * _EW
    ones16 = jnp.ones((_L,), jnp.float32)

    def chunk(i, _):
        base = base0 + i * _C
        pltpu.sync_copy(dst_hbm.at[pl.ds(base, _C)], dix)
        pltpu.sync_copy(src_hbm.at[pl.ds(base, _C)], six)
        cp_u = pltpu.async_copy(u_hbm.at[dix], rows_u, sem_u)
        cp_v = pltpu.async_copy(v_hbm.at[six], rows_v, sem_v)
        for g in range(_C // _L):
            dv = dix[pl.ds(g * _L, _L)]
            sv = six[pl.ds(g * _L, _L)]
            plsc.addupdate_scatter(histd, [dv], ones16)
            plsc.addupdate_scatter(hists, [sv], ones16)
        cp_u.wait()
        cp_v.wait()
        pltpu.sync_copy(rows_u, ud_hbm.at[pl.ds(base, _C)])
        pltpu.sync_copy(rows_v, vs_hbm.at[pl.ds(base, _C)])
        pltpu.sync_copy(rows_v, p_sh.at[dix], add=True)
        return 0

    lax.fori_loop(0, _NCHUNK, chunk, 0)

    plsc.subcore_barrier()

    pltpu.sync_copy(p_sh.at[pl.ds(row0, _NSTRIPE)], p_hbm.at[c, pl.ds(row0, _NSTRIPE)])
    pltpu.sync_copy(histd, histd_hbm.at[wid])
    pltpu.sync_copy(hists, hists_hbm.at[wid])


def _sc_gather(u, v, src, dst):
    mesh = plsc.VectorSubcoreMesh(core_axis_name="c", subcore_axis_name="s")
    fn = functools.partial(
        pl.kernel,
        mesh=mesh,
        compiler_params=pltpu.CompilerParams(needs_layout_passes=False),
        out_type=[
            jax.ShapeDtypeStruct((_E, _D), jnp.float32),
            jax.ShapeDtypeStruct((_E, _D), jnp.float32),
            jax.ShapeDtypeStruct((_NC, _NP, _D), jnp.float32),
            jax.ShapeDtypeStruct((_NW, _NP), jnp.float32),
            jax.ShapeDtypeStruct((_NW, _NP), jnp.float32),
        ],
        scratch_types=[
            pltpu.VMEM((_C,), jnp.int32),
            pltpu.VMEM((_C,), jnp.int32),
            pltpu.VMEM((_C, _D), jnp.float32),
            pltpu.VMEM((_C, _D), jnp.float32),
            pltpu.VMEM((_NP,), jnp.float32),
            pltpu.VMEM((_NP,), jnp.float32),
            pltpu.VMEM_SHARED((_NP, _D), jnp.float32),
            pltpu.SemaphoreType.DMA,
            pltpu.SemaphoreType.DMA,
        ],
    )(_sc_gather_body)
    return fn(u, v, src, dst)


_FW = _D // _NW
_CH = 640
_NCHR = _E // _CH
_GK = 4
_EB4 = _CH * _GK


def _group_fix_b(acc, dstb, vals, sj, tj, jv, b, row, off, col):
    dstv = dstb[b, pl.ds(off, _L)]
    val = vals[b, row, pl.ds(col, _L)]
    val = val * sj + tj
    val = jnp.maximum(val, 0.2 * val)
    cur = plsc.load_gather(acc, [jv, dstv])

    def cond(mm):
        return jnp.any(mm)

    def body(mm):
        plsc.store_scatter(acc, [jv, dstv], val, mask=mm)
        cur2 = plsc.load_gather(acc, [jv, dstv])
        return val > cur2

    lax.while_loop(cond, body, val > cur)


_UNROLL = 4


def _sc_scatter_max_body(y2tr_hbm, dst_hbm, srep_hbm, trep_hbm,
                         agg_hbm,
                         dstb, idxb, vals, st_v, acc,
                         sem_d0, sem_d1, sem_g0, sem_g1):
    c = lax.axis_index("c")
    s = lax.axis_index("s")
    wid = s * _NC + c
    f0 = wid * _FW

    neg_inf = jnp.full((_L,), -jnp.inf, jnp.float32)

    def fill_acc(i, _):
        for j in range(_FW):
            acc[j, pl.ds(i * _L, _L)] = neg_inf
        return 0

    lax.fori_loop(0, _NP // _L, fill_acc, 0)

    pltpu.sync_copy(srep_hbm.at[wid], st_v.at[0])
    pltpu.sync_copy(trep_hbm.at[wid], st_v.at[1])

    lanes = lax.iota(jnp.int32, _L)
    jlane = lax.shift_right_logical(lanes, 2)
    ilane = lanes & 3
    sem_d = (sem_d0, sem_d1)
    sem_g = (sem_g0, sem_g1)
    nchunk = _NCHR // _GK

    def issue(kk, b):
        c0 = kk * _GK
        e0 = c0 * _CH
        pltpu.async_copy(dst_hbm.at[pl.ds(e0, _EB4)], dstb.at[b], sem_d[b])
        idxb[b, :] = (f0 + jlane) * _NCHR + c0 + ilane
        pltpu.async_copy(y2tr_hbm.at[idxb.at[b]], vals.at[b], sem_g[b])

    issue(0, 0)

    def chunk(kk, _):
        for b in range(2):
            @pl.when(kk & 1 == b)
            def _():
                pltpu.make_async_copy(dst_hbm.at[pl.ds(0, _EB4)],
                                      dstb.at[b], sem_d[b]).wait()
                pltpu.make_async_copy(y2tr_hbm.at[idxb.at[b]],
                                      vals.at[b], sem_g[b]).wait()

                @pl.when(kk + 1 < nchunk)
                def _():
                    issue(kk + 1, 1 - b)

                for j in range(_FW):
                    sj = st_v[0, j, pl.ds(0, _L)]
                    tj = st_v[1, j, pl.ds(0, _L)]
                    jv = jnp.full((_L,), j, jnp.int32)
                    row = j * _GK

                    for i in range(_GK):
                        def step(t, _):
                            offs, dsts, vs = [], [], []
                            for q in range(_UNROLL):
                                off = i * _CH + t * (_UNROLL * _L) + q * _L
                                col = t * (_UNROLL * _L) + q * _L
                                offs.append((off, col))
                                dstv = dstb[b, pl.ds(off, _L)]
                                val = vals[b, row + i, pl.ds(col, _L)]
                                val = val * sj + tj
                                val = jnp.maximum(val, 0.2 * val)
                                dsts.append(dstv)
                                vs.append(val)
                            curs = [plsc.load_gather(acc, [jv, d]) for d in dsts]
                            for q in range(_UNROLL):
                                plsc.store_scatter(acc, [jv, dsts[q]], vs[q],
                                                   mask=vs[q] > curs[q])
                            lost = None
                            for q in range(_UNROLL):
                                cur2 = plsc.load_gather(acc, [jv, dsts[q]])
                                lq = vs[q] > cur2
                                lost = lq if lost is None else jnp.logical_or(lost, lq)

                            @pl.when(jnp.any(lost))
                            def _():
                                for off, col in offs:
                                    _group_fix_b(acc, dstb, vals, sj, tj, jv,
                                                 b, row + i, off, col)
                            return 0

                        lax.fori_loop(0, _CH // (_UNROLL * _L), step, 0)
        return 0

    lax.fori_loop(0, nchunk, chunk, 0)

    pltpu.sync_copy(acc, agg_hbm.at[wid])


def _sc_scatter_max(y2tr, dst, srep, trep):
    mesh = plsc.VectorSubcoreMesh(core_axis_name="c", subcore_axis_name="s")
    fn = functools.partial(
        pl.kernel,
        mesh=mesh,
        compiler_params=pltpu.CompilerParams(needs_layout_passes=False),
        out_type=[
            jax.ShapeDtypeStruct((_NW, _FW, _NP), jnp.float32),
        ],
        scratch_types=[
            pltpu.VMEM((2, _EB4), jnp.int32),
            pltpu.VMEM((2, _L), jnp.int32),
            pltpu.VMEM((2, _L, _CH), jnp.float32),
            pltpu.VMEM((2, _FW, _D), jnp.float32),
            pltpu.VMEM((_FW, _NP), jnp.float32),
            pltpu.SemaphoreType.DMA,
            pltpu.SemaphoreType.DMA,
            pltpu.SemaphoreType.DMA,
            pltpu.SemaphoreType.DMA,
        ],
    )(_sc_scatter_max_body)
    return fn(y2tr, dst, srep, trep)[0]


_BN0 = 2000
_BE = 1280


def _k0_body(x_ref, wa_ref, wb_ref, u_ref, v_ref):
    x = x_ref[...]
    u_ref[...] = jnp.dot(x, wa_ref[...], preferred_element_type=jnp.float32)
    v_ref[...] = jnp.dot(x, wb_ref[...], preferred_element_type=jnp.float32)


def _k0(x, wa, wb):
    return pl.pallas_call(
        _k0_body,
        grid=(_N // _BN0,),
        in_specs=[
            pl.BlockSpec((_BN0, _D), lambda i: (i, 0)),
            pl.BlockSpec((_D, _D), lambda i: (0, 0)),
            pl.BlockSpec((_D, _D), lambda i: (0, 0)),
        ],
        out_specs=[
            pl.BlockSpec((_BN0, _D), lambda i: (i, 0)),
            pl.BlockSpec((_BN0, _D), lambda i: (i, 0)),
        ],
        out_shape=[
            jax.ShapeDtypeStruct((_N, _D), jnp.float32),
            jax.ShapeDtypeStruct((_N, _D), jnp.float32),
        ],
    )(x, wa, wb)


def _k3_body(ud_ref, vs_ref, s1_ref, t1_ref, w2_ref, y2t_ref, ss_ref, sq_ref):
    y1 = ud_ref[...] + vs_ref[...]
    h2 = y1 * s1_ref[...] + t1_ref[...]
    h2 = jnp.maximum(h2, 0.2 * h2)
    y2t = jax.lax.dot_general(w2_ref[...], h2, (((1,), (1,)), ((), ())),
                              preferred_element_type=jnp.float32)
    y2t_ref[...] = y2t

    @pl.when(pl.program_id(0) == 0)
    def _():
        ss_ref[...] = jnp.zeros_like(ss_ref)
        sq_ref[...] = jnp.zeros_like(sq_ref)

    ss_ref[...] += jnp.sum(y2t, axis=1)[None, :]
    sq_ref[...] += jnp.sum(y2t * y2t, axis=1)[None, :]


def _k3(ud, vs, s1, t1, w2):
    return pl.pallas_call(
        _k3_body,
        grid=(_E // _BE,),
        in_specs=[
            pl.BlockSpec((_BE, _D), lambda i: (i, 0)),
            pl.BlockSpec((_BE, _D), lambda i: (i, 0)),
            pl.BlockSpec((1, _D), lambda i: (0, 0)),
            pl.BlockSpec((1, _D), lambda i: (0, 0)),
            pl.BlockSpec((_D, _D), lambda i: (0, 0)),
        ],
        out_specs=[
            pl.BlockSpec((_D, _BE), lambda i: (0, i)),
            pl.BlockSpec((1, _D), lambda i: (0, 0)),
            pl.BlockSpec((1, _D), lambda i: (0, 0)),
        ],
        out_shape=[
            jax.ShapeDtypeStruct((_D, _E), jnp.float32),
            jax.ShapeDtypeStruct((1, _D), jnp.float32),
            jax.ShapeDtypeStruct((1, _D), jnp.float32),
        ],
    )(ud, vs, s1, t1, w2)


def _k5_body(agg_ref, x_ref, o_ref):
    a = agg_ref[...].T
    a = jnp.where(jnp.isfinite(a), a, 0.0)
    o = a + x_ref[...]
    o_ref[...] = jnp.maximum(o, 0.2 * o)


_BN5 = 2048


def _k5(agg, xp):
    return pl.pallas_call(
        _k5_body,
        grid=(_NP // _BN5,),
        in_specs=[
            pl.BlockSpec((_D, _BN5), lambda i: (0, i)),
            pl.BlockSpec((_BN5, _D), lambda i: (i, 0)),
        ],
        out_specs=pl.BlockSpec((_BN5, _D), lambda i: (i, 0)),
        out_shape=jax.ShapeDtypeStruct((_NP, _D), jnp.float32),
    )(agg, xp)


def kernel(x, edge_index, W1, g1, b1, W2, g2, b2):
    N, D = x.shape
    E = edge_index.shape[1]
    src = edge_index[0]
    dst = edge_index[1]

    wa = (W1[:, :D] - W1[:, D:]).T
    wb = W1[:, D:].T
    u, v = _k0(x, wa, wb)

    ud, vs, p_parts, histd, hists = _sc_gather(u, v, src, dst)
    P = p_parts[0, :N] + p_parts[1, :N]
    cnt_dst = jnp.sum(histd, axis=0)[:N]
    cnt_src = jnp.sum(hists, axis=0)[:N]

    sum1 = cnt_dst @ u + cnt_src @ v
    sq1 = cnt_dst @ (u * u) + 2.0 * jnp.sum(u * P, axis=0) + cnt_src @ (v * v)
    mean1 = sum1 / E
    var1 = sq1 / E - mean1 * mean1
    s1 = g1 / jnp.sqrt(var1 + _EPS)
    t1 = b1 - mean1 * s1

    y2t, ssum, ssq = _k3(ud, vs, s1[None, :], t1[None, :], W2)
    mean2 = ssum[0] / E
    var2 = ssq[0] / E - mean2 * mean2
    s2 = g2 / jnp.sqrt(var2 + _EPS)
    t2 = b2 - mean2 * s2

    srep = jnp.broadcast_to(s2.reshape(_NW, _FW, 1), (_NW, _FW, _D))
    trep = jnp.broadcast_to(t2.reshape(_NW, _FW, 1), (_NW, _FW, _D))
    y2tr = y2t.reshape(_D * _NCHR, _CH)
    agg = _sc_scatter_max(y2tr, dst, srep, trep).reshape(_D, _NP)

    xp = jnp.pad(x, ((0, _NP - N), (0, 0)))
    return _k5(agg, xp)[:N]


# speedup vs baseline: 2.4491x; 1.0040x over previous
import functools

import jax
import jax.numpy as jnp
from jax import lax
from jax.experimental import pallas as pl
from jax.experimental.pallas import tpu as pltpu
from jax.experimental.pallas import tpu_sc as plsc

_EPS = 1e-5

_N = 10000
_E = 320000
_D = 128

_NC = 2
_NS = 16
_NW = _NC * _NS
_L = 16

_EW = _E // _NW
_C = 80
_NCHUNK = _EW // _C
_NP = 10240
_NSTRIPE = _NP // _NS
_ZR = 80


def _leaky(x):
    return jnp.maximum(x, 0.2 * x)


def _sc_gather_body(u_hbm, v_hbm, src_hbm, dst_hbm,
                    ud_hbm, vs_hbm, p_hbm, histd_hbm, hists_hbm,
                    dix, six, rows_u, rows_v, histd, hists,
                    p_sh, sem_u, sem_v):
    c = lax.axis_index("c")
    s = lax.axis_index("s")
    wid = s * _NC + c

    def fill_zrow(i, _):
        for j in range(_D // _L):
            rows_u[i, pl.ds(j * _L, _L)] = jnp.zeros((_L,), jnp.float32)
        return 0

    lax.fori_loop(0, _ZR, fill_zrow, 0)

    def fill_hist(i, _):
        histd[pl.ds(i * _L, _L)] = jnp.zeros((_L,), jnp.float32)
        hists[pl.ds(i * _L, _L)] = jnp.zeros((_L,), jnp.float32)
        return 0

    lax.fori_loop(0, _NP // _L, fill_hist, 0)

    row0 = s * _NSTRIPE
    for k in range(_NSTRIPE // _ZR):
        pltpu.sync_copy(rows_u, p_sh.at[pl.ds(row0 + k * _ZR, _ZR)])

    plsc.subcore_barrier()

    base0 = wid * _EW
    ones16 = jnp.ones((_L,), jnp.float32)

    def chunk(i, _):
        base = base0 + i * _C
        pltpu.sync_copy(dst_hbm.at[pl.ds(base, _C)], dix)
        pltpu.sync_copy(src_hbm.at[pl.ds(base, _C)], six)
        cp_u = pltpu.async_copy(u_hbm.at[dix], rows_u, sem_u)
        cp_v = pltpu.async_copy(v_hbm.at[six], rows_v, sem_v)
        for g in range(_C // _L):
            dv = dix[pl.ds(g * _L, _L)]
            sv = six[pl.ds(g * _L, _L)]
            plsc.addupdate_scatter(histd, [dv], ones16)
            plsc.addupdate_scatter(hists, [sv], ones16)
        cp_u.wait()
        cp_v.wait()
        pltpu.sync_copy(rows_u, ud_hbm.at[pl.ds(base, _C)])
        pltpu.sync_copy(rows_v, vs_hbm.at[pl.ds(base, _C)])
        pltpu.sync_copy(rows_v, p_sh.at[dix], add=True)
        return 0

    lax.fori_loop(0, _NCHUNK, chunk, 0)

    plsc.subcore_barrier()

    pltpu.sync_copy(p_sh.at[pl.ds(row0, _NSTRIPE)], p_hbm.at[c, pl.ds(row0, _NSTRIPE)])
    pltpu.sync_copy(histd, histd_hbm.at[wid])
    pltpu.sync_copy(hists, hists_hbm.at[wid])


def _sc_gather(u, v, src, dst):
    mesh = plsc.VectorSubcoreMesh(core_axis_name="c", subcore_axis_name="s")
    fn = functools.partial(
        pl.kernel,
        mesh=mesh,
        compiler_params=pltpu.CompilerParams(needs_layout_passes=False),
        out_type=[
            jax.ShapeDtypeStruct((_E, _D), jnp.float32),
            jax.ShapeDtypeStruct((_E, _D), jnp.float32),
            jax.ShapeDtypeStruct((_NC, _NP, _D), jnp.float32),
            jax.ShapeDtypeStruct((_NW, _NP), jnp.float32),
            jax.ShapeDtypeStruct((_NW, _NP), jnp.float32),
        ],
        scratch_types=[
            pltpu.VMEM((_C,), jnp.int32),
            pltpu.VMEM((_C,), jnp.int32),
            pltpu.VMEM((_C, _D), jnp.float32),
            pltpu.VMEM((_C, _D), jnp.float32),
            pltpu.VMEM((_NP,), jnp.float32),
            pltpu.VMEM((_NP,), jnp.float32),
            pltpu.VMEM_SHARED((_NP, _D), jnp.float32),
            pltpu.SemaphoreType.DMA,
            pltpu.SemaphoreType.DMA,
        ],
    )(_sc_gather_body)
    return fn(u, v, src, dst)


_FW = _D // _NW
_CH = 640
_NCHR = _E // _CH
_GK = 4
_EB4 = _CH * _GK


def _group_fix_b(acc, dstb, vals, jv, b, row, off, col):
    dstv = dstb[b, pl.ds(off, _L)]
    val = vals[b, row, pl.ds(col, _L)]
    cur = plsc.load_gather(acc, [jv, dstv])

    def cond(mm):
        return jnp.any(mm)

    def body(mm):
        plsc.store_scatter(acc, [jv, dstv], val, mask=mm)
        cur2 = plsc.load_gather(acc, [jv, dstv])
        return val > cur2

    lax.while_loop(cond, body, val > cur)


_UNROLL = 4


def _sc_scatter_max_body(y2tr_hbm, dst_hbm,
                         agg_hbm,
                         dstb, idxb, vals, acc,
                         sem_d0, sem_d1, sem_g0, sem_g1):
    c = lax.axis_index("c")
    s = lax.axis_index("s")
    wid = s * _NC + c
    f0 = wid * _FW

    neg_inf = jnp.full((_L,), -jnp.inf, jnp.float32)

    def fill_acc(i, _):
        for j in range(_FW):
            acc[j, pl.ds(i * _L, _L)] = neg_inf
        return 0

    lax.fori_loop(0, _NP // _L, fill_acc, 0)

    lanes = lax.iota(jnp.int32, _L)
    jlane = lax.shift_right_logical(lanes, 2)
    ilane = lanes & 3
    sem_d = (sem_d0, sem_d1)
    sem_g = (sem_g0, sem_g1)
    nchunk = _NCHR // _GK

    def issue(kk, b):
        c0 = kk * _GK
        e0 = c0 * _CH
        pltpu.async_copy(dst_hbm.at[pl.ds(e0, _EB4)], dstb.at[b], sem_d[b])
        idxb[b, :] = (f0 + jlane) * _NCHR + c0 + ilane
        pltpu.async_copy(y2tr_hbm.at[idxb.at[b]], vals.at[b], sem_g[b])

    issue(0, 0)

    def chunk(kk, _):
        for b in range(2):
            @pl.when(kk & 1 == b)
            def _():
                pltpu.make_async_copy(dst_hbm.at[pl.ds(0, _EB4)],
                                      dstb.at[b], sem_d[b]).wait()
                pltpu.make_async_copy(y2tr_hbm.at[idxb.at[b]],
                                      vals.at[b], sem_g[b]).wait()

                @pl.when(kk + 1 < nchunk)
                def _():
                    issue(kk + 1, 1 - b)

                for j in range(_FW):
                    jv = jnp.full((_L,), j, jnp.int32)
                    row = j * _GK

                    for i in range(_GK):
                        def step(t, _):
                            offs, dsts, vs = [], [], []
                            for q in range(_UNROLL):
                                off = i * _CH + t * (_UNROLL * _L) + q * _L
                                col = t * (_UNROLL * _L) + q * _L
                                offs.append((off, col))
                                dstv = dstb[b, pl.ds(off, _L)]
                                val = vals[b, row + i, pl.ds(col, _L)]
                                dsts.append(dstv)
                                vs.append(val)
                            curs = [plsc.load_gather(acc, [jv, d]) for d in dsts]
                            for q in range(_UNROLL):
                                plsc.store_scatter(acc, [jv, dsts[q]], vs[q],
                                                   mask=vs[q] > curs[q])
                            lost = None
                            for q in range(_UNROLL):
                                cur2 = plsc.load_gather(acc, [jv, dsts[q]])
                                lq = vs[q] > cur2
                                lost = lq if lost is None else jnp.logical_or(lost, lq)

                            @pl.when(jnp.any(lost))
                            def _():
                                for off, col in offs:
                                    _group_fix_b(acc, dstb, vals, jv,
                                                 b, row + i, off, col)
                            return 0

                        lax.fori_loop(0, _CH // (_UNROLL * _L), step, 0)
        return 0

    lax.fori_loop(0, nchunk, chunk, 0)

    pltpu.sync_copy(acc, agg_hbm.at[wid])


def _sc_scatter_max(y2tr, dst):
    mesh = plsc.VectorSubcoreMesh(core_axis_name="c", subcore_axis_name="s")
    fn = functools.partial(
        pl.kernel,
        mesh=mesh,
        compiler_params=pltpu.CompilerParams(needs_layout_passes=False),
        out_type=[
            jax.ShapeDtypeStruct((_NW, _FW, _NP), jnp.float32),
        ],
        scratch_types=[
            pltpu.VMEM((2, _EB4), jnp.int32),
            pltpu.VMEM((2, _L), jnp.int32),
            pltpu.VMEM((2, _L, _CH), jnp.float32),
            pltpu.VMEM((_FW, _NP), jnp.float32),
            pltpu.SemaphoreType.DMA,
            pltpu.SemaphoreType.DMA,
            pltpu.SemaphoreType.DMA,
            pltpu.SemaphoreType.DMA,
        ],
    )(_sc_scatter_max_body)
    return fn(y2tr, dst)[0]


_BN0 = 2000
_BE = 1280


def _k0_body(x_ref, wa_ref, wb_ref, u_ref, v_ref):
    x = x_ref[...]
    u_ref[...] = jnp.dot(x, wa_ref[...], preferred_element_type=jnp.float32)
    v_ref[...] = jnp.dot(x, wb_ref[...], preferred_element_type=jnp.float32)


def _k0(x, wa, wb):
    return pl.pallas_call(
        _k0_body,
        grid=(_N // _BN0,),
        in_specs=[
            pl.BlockSpec((_BN0, _D), lambda i: (i, 0)),
            pl.BlockSpec((_D, _D), lambda i: (0, 0)),
            pl.BlockSpec((_D, _D), lambda i: (0, 0)),
        ],
        out_specs=[
            pl.BlockSpec((_BN0, _D), lambda i: (i, 0)),
            pl.BlockSpec((_BN0, _D), lambda i: (i, 0)),
        ],
        out_shape=[
            jax.ShapeDtypeStruct((_N, _D), jnp.float32),
            jax.ShapeDtypeStruct((_N, _D), jnp.float32),
        ],
    )(x, wa, wb)


def _k3_body(ud_ref, vs_ref, s1_ref, t1_ref, w2_ref, y2t_ref, ss_ref, sq_ref):
    y1 = ud_ref[...] + vs_ref[...]
    h2 = y1 * s1_ref[...] + t1_ref[...]
    h2 = jnp.maximum(h2, 0.2 * h2)
    y2t = jax.lax.dot_general(w2_ref[...], h2, (((1,), (1,)), ((), ())),
                              preferred_element_type=jnp.float32)
    y2t_ref[...] = y2t

    @pl.when(pl.program_id(0) == 0)
    def _():
        ss_ref[...] = jnp.zeros_like(ss_ref)
        sq_ref[...] = jnp.zeros_like(sq_ref)

    ss_ref[...] += jnp.sum(y2t, axis=1)[None, :]
    sq_ref[...] += jnp.sum(y2t * y2t, axis=1)[None, :]


def _k3(ud, vs, s1, t1, w2):
    return pl.pallas_call(
        _k3_body,
        grid=(_E // _BE,),
        in_specs=[
            pl.BlockSpec((_BE, _D), lambda i: (i, 0)),
            pl.BlockSpec((_BE, _D), lambda i: (i, 0)),
            pl.BlockSpec((1, _D), lambda i: (0, 0)),
            pl.BlockSpec((1, _D), lambda i: (0, 0)),
            pl.BlockSpec((_D, _D), lambda i: (0, 0)),
        ],
        out_specs=[
            pl.BlockSpec((_D, _BE), lambda i: (0, i)),
            pl.BlockSpec((1, _D), lambda i: (0, 0)),
            pl.BlockSpec((1, _D), lambda i: (0, 0)),
        ],
        out_shape=[
            jax.ShapeDtypeStruct((_D, _E), jnp.float32),
            jax.ShapeDtypeStruct((1, _D), jnp.float32),
            jax.ShapeDtypeStruct((1, _D), jnp.float32),
        ],
    )(ud, vs, s1, t1, w2)


def _k5_body(agg_ref, x_ref, c1_ref, t2_ref, o_ref):
    a = agg_ref[...].T
    h = a * c1_ref[...] + t2_ref[...]
    h = jnp.maximum(h, 0.2 * h)
    h = jnp.where(jnp.isfinite(a), h, 0.0)
    o = h + x_ref[...]
    o_ref[...] = jnp.maximum(o, 0.2 * o)


_BN5 = 2048


def _k5(agg, xp, c1, t2):
    return pl.pallas_call(
        _k5_body,
        grid=(_NP // _BN5,),
        in_specs=[
            pl.BlockSpec((_D, _BN5), lambda i: (0, i)),
            pl.BlockSpec((_BN5, _D), lambda i: (i, 0)),
            pl.BlockSpec((1, _D), lambda i: (0, 0)),
            pl.BlockSpec((1, _D), lambda i: (0, 0)),
        ],
        out_specs=pl.BlockSpec((_BN5, _D), lambda i: (i, 0)),
        out_shape=jax.ShapeDtypeStruct((_NP, _D), jnp.float32),
    )(agg, xp, c1, t2)


def kernel(x, edge_index, W1, g1, b1, W2, g2, b2):
    N, D = x.shape
    E = edge_index.shape[1]
    src = edge_index[0]
    dst = edge_index[1]

    wa = (W1[:, :D] - W1[:, D:]).T
    wb = W1[:, D:].T
    u, v = _k0(x, wa, wb)

    ud, vs, p_parts, histd, hists = _sc_gather(u, v, src, dst)
    P = p_parts[0, :N] + p_parts[1, :N]
    cnt_dst = jnp.sum(histd, axis=0)[:N]
    cnt_src = jnp.sum(hists, axis=0)[:N]

    sum1 = cnt_dst @ u + cnt_src @ v
    sq1 = cnt_dst @ (u * u) + 2.0 * jnp.sum(u * P, axis=0) + cnt_src @ (v * v)
    mean1 = sum1 / E
    var1 = sq1 / E - mean1 * mean1
    s1 = g1 / jnp.sqrt(var1 + _EPS)
    t1 = b1 - mean1 * s1

    sgn = jnp.where(g2 >= 0, 1.0, -1.0).astype(jnp.float32)
    y2t, ssum, ssq = _k3(ud, vs, s1[None, :], t1[None, :], W2 * sgn[:, None])
    mean2 = sgn * ssum[0] / E
    var2 = ssq[0] / E - mean2 * mean2
    s2 = g2 / jnp.sqrt(var2 + _EPS)
    t2 = b2 - mean2 * s2

    y2tr = y2t.reshape(_D * _NCHR, _CH)
    agg = _sc_scatter_max(y2tr, dst).reshape(_D, _NP)

    xp = jnp.pad(x, ((0, _NP - N), (0, 0)))
    c1 = (s2 * sgn)[None, :]
    return _k5(agg, xp, c1, t2[None, :])[:N]


# speedup vs baseline: 2.6542x; 1.0837x over previous
import functools

import jax
import jax.numpy as jnp
from jax import lax
from jax.experimental import pallas as pl
from jax.experimental.pallas import tpu as pltpu
from jax.experimental.pallas import tpu_sc as plsc

_EPS = 1e-5

_N = 10000
_E = 320000
_D = 128

_NC = 2
_NS = 16
_NW = _NC * _NS
_L = 16

_EW = _E // _NW
_C = 40
_NCHUNK = _EW // _C
_NP = 10240
_NSTRIPE = _NP // _NS
_ZR = 40


def _leaky(x):
    return jnp.maximum(x, 0.2 * x)


def _sc_gather_body(u_hbm, v_hbm, src_hbm, dst_hbm,
                    ud_hbm, vs_hbm, p_hbm, histd_hbm, hists_hbm,
                    dix, six, rows_u, rows_v, histd, hists, p_sh,
                    sem_di0, sem_di1, sem_si0, sem_si1,
                    sem_gu0, sem_gu1, sem_gv0, sem_gv1, sem_w0, sem_w1):
    c = lax.axis_index("c")
    s = lax.axis_index("s")
    wid = s * _NC + c
    sem_di = (sem_di0, sem_di1)
    sem_si = (sem_si0, sem_si1)
    sem_gu = (sem_gu0, sem_gu1)
    sem_gv = (sem_gv0, sem_gv1)
    sem_w = (sem_w0, sem_w1)

    def fill_zrow(i, _):
        for j in range(_D // _L):
            rows_u[0, i, pl.ds(j * _L, _L)] = jnp.zeros((_L,), jnp.float32)
        return 0

    lax.fori_loop(0, _ZR, fill_zrow, 0)

    def fill_hist(i, _):
        histd[pl.ds(i * _L, _L)] = jnp.zeros((_L,), jnp.float32)
        hists[pl.ds(i * _L, _L)] = jnp.zeros((_L,), jnp.float32)
        return 0

    lax.fori_loop(0, _NP // _L, fill_hist, 0)

    row0 = s * _NSTRIPE
    for k in range(_NSTRIPE // _ZR):
        pltpu.sync_copy(rows_u.at[0], p_sh.at[pl.ds(row0 + k * _ZR, _ZR)])

    plsc.subcore_barrier()

    base0 = wid * _EW
    ones16 = jnp.ones((_L,), jnp.float32)

    def issue_idx(i, b):
        base = base0 + i * _C
        pltpu.async_copy(dst_hbm.at[pl.ds(base, _C)], dix.at[b], sem_di[b])
        pltpu.async_copy(src_hbm.at[pl.ds(base, _C)], six.at[b], sem_si[b])

    def wait_idx(b):
        pltpu.make_async_copy(dst_hbm.at[pl.ds(0, _C)], dix.at[b], sem_di[b]).wait()
        pltpu.make_async_copy(src_hbm.at[pl.ds(0, _C)], six.at[b], sem_si[b]).wait()

    def issue_gather(b):
        pltpu.async_copy(u_hbm.at[dix.at[b]], rows_u.at[b], sem_gu[b])
        pltpu.async_copy(v_hbm.at[six.at[b]], rows_v.at[b], sem_gv[b])

    def wait_gather(b):
        pltpu.make_async_copy(u_hbm.at[dix.at[b]], rows_u.at[b], sem_gu[b]).wait()
        pltpu.make_async_copy(v_hbm.at[six.at[b]], rows_v.at[b], sem_gv[b]).wait()

    def wait_writes(b):
        pltpu.make_async_copy(rows_u.at[b], ud_hbm.at[pl.ds(0, _C)], sem_w[b]).wait()
        pltpu.make_async_copy(rows_v.at[b], vs_hbm.at[pl.ds(0, _C)], sem_w[b]).wait()

    issue_idx(0, 0)
    issue_idx(1, 1)
    wait_idx(0)
    issue_gather(0)

    def chunk(i, _):
        for b in range(2):
            @pl.when(i & 1 == b)
            def _():
                base = base0 + i * _C
                wait_gather(b)
                for g in range(_C // _L):
                    dv = dix[b, pl.ds(g * _L, _L)]
                    sv = six[b, pl.ds(g * _L, _L)]
                    plsc.addupdate_scatter(histd, [dv], ones16)
                    plsc.addupdate_scatter(hists, [sv], ones16)

                @pl.when(i + 1 < _NCHUNK)
                def _():
                    wait_idx(1 - b)

                    @pl.when(i >= 1)
                    def _():
                        wait_writes(1 - b)

                    issue_gather(1 - b)

                pltpu.async_copy(rows_u.at[b], ud_hbm.at[pl.ds(base, _C)], sem_w[b])
                pltpu.async_copy(rows_v.at[b], vs_hbm.at[pl.ds(base, _C)], sem_w[b])
                pltpu.sync_copy(rows_v.at[b], p_sh.at[dix.at[b]], add=True)

                @pl.when(i + 2 < _NCHUNK)
                def _():
                    issue_idx(i + 2, b)
        return 0

    lax.fori_loop(0, _NCHUNK, chunk, 0)

    wait_writes(0)
    wait_writes(1)

    plsc.subcore_barrier()

    pltpu.sync_copy(p_sh.at[pl.ds(row0, _NSTRIPE)], p_hbm.at[c, pl.ds(row0, _NSTRIPE)])
    pltpu.sync_copy(histd, histd_hbm.at[wid])
    pltpu.sync_copy(hists, hists_hbm.at[wid])


def _sc_gather(u, v, src, dst):
    mesh = plsc.VectorSubcoreMesh(core_axis_name="c", subcore_axis_name="s")
    fn = functools.partial(
        pl.kernel,
        mesh=mesh,
        compiler_params=pltpu.CompilerParams(needs_layout_passes=False),
        out_type=[
            jax.ShapeDtypeStruct((_E, _D), jnp.float32),
            jax.ShapeDtypeStruct((_E, _D), jnp.float32),
            jax.ShapeDtypeStruct((_NC, _NP, _D), jnp.float32),
            jax.ShapeDtypeStruct((_NW, _NP), jnp.float32),
            jax.ShapeDtypeStruct((_NW, _NP), jnp.float32),
        ],
        scratch_types=[
            pltpu.VMEM((2, _C), jnp.int32),
            pltpu.VMEM((2, _C), jnp.int32),
            pltpu.VMEM((2, _C, _D), jnp.float32),
            pltpu.VMEM((2, _C, _D), jnp.float32),
            pltpu.VMEM((_NP,), jnp.float32),
            pltpu.VMEM((_NP,), jnp.float32),
            pltpu.VMEM_SHARED((_NP, _D), jnp.float32),
        ] + [pltpu.SemaphoreType.DMA] * 10,
    )(_sc_gather_body)
    return fn(u, v, src, dst)


_FW = _D // _NW
_CH = 640
_NCHR = _E // _CH
_GK = 4
_EB4 = _CH * _GK


def _group_fix_b(acc, dstb, vals, jv, b, row, off, col):
    dstv = dstb[b, pl.ds(off, _L)]
    val = vals[b, row, pl.ds(col, _L)]
    cur = plsc.load_gather(acc, [jv, dstv])

    def cond(mm):
        return jnp.any(mm)

    def body(mm):
        plsc.store_scatter(acc, [jv, dstv], val, mask=mm)
        cur2 = plsc.load_gather(acc, [jv, dstv])
        return val > cur2

    lax.while_loop(cond, body, val > cur)


_UNROLL = 4


def _sc_scatter_max_body(y2tr_hbm, dst_hbm,
                         agg_hbm,
                         dstb, idxb, vals, acc,
                         sem_d0, sem_d1, sem_g0, sem_g1):
    c = lax.axis_index("c")
    s = lax.axis_index("s")
    wid = s * _NC + c
    f0 = wid * _FW

    neg_inf = jnp.full((_L,), -jnp.inf, jnp.float32)

    def fill_acc(i, _):
        for j in range(_FW):
            acc[j, pl.ds(i * _L, _L)] = neg_inf
        return 0

    lax.fori_loop(0, _NP // _L, fill_acc, 0)

    lanes = lax.iota(jnp.int32, _L)
    jlane = lax.shift_right_logical(lanes, 2)
    ilane = lanes & 3
    sem_d = (sem_d0, sem_d1)
    sem_g = (sem_g0, sem_g1)
    nchunk = _NCHR // _GK

    def issue(kk, b):
        c0 = kk * _GK
        e0 = c0 * _CH
        pltpu.async_copy(dst_hbm.at[pl.ds(e0, _EB4)], dstb.at[b], sem_d[b])
        idxb[b, :] = (c0 + ilane) * _D + f0 + jlane
        pltpu.async_copy(y2tr_hbm.at[idxb.at[b]], vals.at[b], sem_g[b])

    issue(0, 0)

    def chunk(kk, _):
        for b in range(2):
            @pl.when(kk & 1 == b)
            def _():
                pltpu.make_async_copy(dst_hbm.at[pl.ds(0, _EB4)],
                                      dstb.at[b], sem_d[b]).wait()
                pltpu.make_async_copy(y2tr_hbm.at[idxb.at[b]],
                                      vals.at[b], sem_g[b]).wait()

                @pl.when(kk + 1 < nchunk)
                def _():
                    issue(kk + 1, 1 - b)

                for j in range(_FW):
                    jv = jnp.full((_L,), j, jnp.int32)
                    row = j * _GK

                    for i in range(_GK):
                        def step(t, _):
                            offs, dsts, vs = [], [], []
                            for q in range(_UNROLL):
                                off = i * _CH + t * (_UNROLL * _L) + q * _L
                                col = t * (_UNROLL * _L) + q * _L
                                offs.append((off, col))
                                dstv = dstb[b, pl.ds(off, _L)]
                                val = vals[b, row + i, pl.ds(col, _L)]
                                dsts.append(dstv)
                                vs.append(val)
                            curs = [plsc.load_gather(acc, [jv, d]) for d in dsts]
                            for q in range(_UNROLL):
                                plsc.store_scatter(acc, [jv, dsts[q]], vs[q],
                                                   mask=vs[q] > curs[q])
                            lost = None
                            for q in range(_UNROLL):
                                cur2 = plsc.load_gather(acc, [jv, dsts[q]])
                                lq = vs[q] > cur2
                                lost = lq if lost is None else jnp.logical_or(lost, lq)

                            @pl.when(jnp.any(lost))
                            def _():
                                for off, col in offs:
                                    _group_fix_b(acc, dstb, vals, jv,
                                                 b, row + i, off, col)
                            return 0

                        lax.fori_loop(0, _CH // (_UNROLL * _L), step, 0)
        return 0

    lax.fori_loop(0, nchunk, chunk, 0)

    pltpu.sync_copy(acc, agg_hbm.at[wid])


def _sc_scatter_max(y2tr, dst):
    mesh = plsc.VectorSubcoreMesh(core_axis_name="c", subcore_axis_name="s")
    fn = functools.partial(
        pl.kernel,
        mesh=mesh,
        compiler_params=pltpu.CompilerParams(needs_layout_passes=False),
        out_type=[
            jax.ShapeDtypeStruct((_NW, _FW, _NP), jnp.float32),
        ],
        scratch_types=[
            pltpu.VMEM((2, _EB4), jnp.int32),
            pltpu.VMEM((2, _L), jnp.int32),
            pltpu.VMEM((2, _L, _CH), jnp.float32),
            pltpu.VMEM((_FW, _NP), jnp.float32),
            pltpu.SemaphoreType.DMA,
            pltpu.SemaphoreType.DMA,
            pltpu.SemaphoreType.DMA,
            pltpu.SemaphoreType.DMA,
        ],
    )(_sc_scatter_max_body)
    return fn(y2tr, dst)[0]


_BN0 = 2000
_BE = _CH


def _k0_body(x_ref, wa_ref, wb_ref, u_ref, v_ref):
    x = x_ref[...]
    u_ref[...] = jnp.dot(x, wa_ref[...], preferred_element_type=jnp.float32)
    v_ref[...] = jnp.dot(x, wb_ref[...], preferred_element_type=jnp.float32)


def _k0(x, wa, wb):
    return pl.pallas_call(
        _k0_body,
        grid=(_N // _BN0,),
        in_specs=[
            pl.BlockSpec((_BN0, _D), lambda i: (i, 0)),
            pl.BlockSpec((_D, _D), lambda i: (0, 0)),
            pl.BlockSpec((_D, _D), lambda i: (0, 0)),
        ],
        out_specs=[
            pl.BlockSpec((_BN0, _D), lambda i: (i, 0)),
            pl.BlockSpec((_BN0, _D), lambda i: (i, 0)),
        ],
        out_shape=[
            jax.ShapeDtypeStruct((_N, _D), jnp.float32),
            jax.ShapeDtypeStruct((_N, _D), jnp.float32),
        ],
    )(x, wa, wb)


def _k3_body(ud_ref, vs_ref, s1_ref, t1_ref, w2_ref, y2t_ref, ss_ref, sq_ref):
    y1 = ud_ref[...] + vs_ref[...]
    h2 = y1 * s1_ref[...] + t1_ref[...]
    h2 = jnp.maximum(h2, 0.2 * h2)
    y2t = jax.lax.dot_general(w2_ref[...], h2, (((1,), (1,)), ((), ())),
                              preferred_element_type=jnp.float32)
    y2t_ref[0] = y2t

    @pl.when(pl.program_id(0) == 0)
    def _():
        ss_ref[...] = jnp.zeros_like(ss_ref)
        sq_ref[...] = jnp.zeros_like(sq_ref)

    ss_ref[...] += jnp.sum(y2t, axis=1)[None, :]
    sq_ref[...] += jnp.sum(y2t * y2t, axis=1)[None, :]


def _k3(ud, vs, s1, t1, w2):
    return pl.pallas_call(
        _k3_body,
        grid=(_E // _BE,),
        in_specs=[
            pl.BlockSpec((_BE, _D), lambda i: (i, 0)),
            pl.BlockSpec((_BE, _D), lambda i: (i, 0)),
            pl.BlockSpec((1, _D), lambda i: (0, 0)),
            pl.BlockSpec((1, _D), lambda i: (0, 0)),
            pl.BlockSpec((_D, _D), lambda i: (0, 0)),
        ],
        out_specs=[
            pl.BlockSpec((1, _D, _BE), lambda i: (i, 0, 0)),
            pl.BlockSpec((1, _D), lambda i: (0, 0)),
            pl.BlockSpec((1, _D), lambda i: (0, 0)),
        ],
        out_shape=[
            jax.ShapeDtypeStruct((_NCHR, _D, _CH), jnp.float32),
            jax.ShapeDtypeStruct((1, _D), jnp.float32),
            jax.ShapeDtypeStruct((1, _D), jnp.float32),
        ],
    )(ud, vs, s1, t1, w2)


def _k5_body(agg_ref, x_ref, c1_ref, t2_ref, o_ref):
    a = agg_ref[...].T
    h = a * c1_ref[...] + t2_ref[...]
    h = jnp.maximum(h, 0.2 * h)
    h = jnp.where(jnp.isfinite(a), h, 0.0)
    o = h + x_ref[...]
    o_ref[...] = jnp.maximum(o, 0.2 * o)


_BN5 = 2048


def _k5(agg, xp, c1, t2):
    return pl.pallas_call(
        _k5_body,
        grid=(_NP // _BN5,),
        in_specs=[
            pl.BlockSpec((_D, _BN5), lambda i: (0, i)),
            pl.BlockSpec((_BN5, _D), lambda i: (i, 0)),
            pl.BlockSpec((1, _D), lambda i: (0, 0)),
            pl.BlockSpec((1, _D), lambda i: (0, 0)),
        ],
        out_specs=pl.BlockSpec((_BN5, _D), lambda i: (i, 0)),
        out_shape=jax.ShapeDtypeStruct((_NP, _D), jnp.float32),
    )(agg, xp, c1, t2)


def kernel(x, edge_index, W1, g1, b1, W2, g2, b2):
    N, D = x.shape
    E = edge_index.shape[1]
    src = edge_index[0]
    dst = edge_index[1]

    wa = (W1[:, :D] - W1[:, D:]).T
    wb = W1[:, D:].T
    u, v = _k0(x, wa, wb)

    ud, vs, p_parts, histd, hists = _sc_gather(u, v, src, dst)
    P = p_parts[0, :N] + p_parts[1, :N]
    cnt_dst = jnp.sum(histd, axis=0)[:N]
    cnt_src = jnp.sum(hists, axis=0)[:N]

    sum1 = cnt_dst @ u + cnt_src @ v
    sq1 = cnt_dst @ (u * u) + 2.0 * jnp.sum(u * P, axis=0) + cnt_src @ (v * v)
    mean1 = sum1 / E
    var1 = sq1 / E - mean1 * mean1
    s1 = g1 / jnp.sqrt(var1 + _EPS)
    t1 = b1 - mean1 * s1

    sgn = jnp.where(g2 >= 0, 1.0, -1.0).astype(jnp.float32)
    y2t, ssum, ssq = _k3(ud, vs, s1[None, :], t1[None, :], W2 * sgn[:, None])
    mean2 = sgn * ssum[0] / E
    var2 = ssq[0] / E - mean2 * mean2
    s2 = g2 / jnp.sqrt(var2 + _EPS)
    t2 = b2 - mean2 * s2

    y2tr = y2t.reshape(_NCHR * _D, _CH)
    agg = _sc_scatter_max(y2tr, dst).reshape(_D, _NP)

    xp = jnp.pad(x, ((0, _NP - N), (0, 0)))
    c1 = (s2 * sgn)[None, :]
    return _k5(agg, xp, c1, t2[None, :])[:N]
